# baseline (speedup 1.0000x reference)
import functools

import jax
import jax.numpy as jnp
from jax.experimental import pallas as pl
from jax.experimental.pallas import tpu as pltpu


def _fused_body(t_spans, seq, hh, with_proj, spans_ref,
                hid0_ref, hid1_ref, w1_ref, w2_ref, pe_ref, ba_ref,
                wout_ref, bout_ref, *refs):
    if with_proj:
        probs_ref, proj_ref = refs
        pooled_ref = None
    else:
        probs_ref, pooled_ref = refs
        proj_ref = None
    b = pl.program_id(0)
    hid0 = hid0_ref[0]
    hid1 = hid1_ref[0]
    sc = jax.lax.dot_general(w2_ref[:, :hh], hid0, (((1,), (1,)), ((), ())),
                             preferred_element_type=jnp.float32)
    sc = sc + jax.lax.dot_general(w2_ref[:, hh:], hid1,
                                  (((1,), (1,)), ((), ())),
                                  preferred_element_type=jnp.float32)
    pedot = jax.lax.dot_general(pe_ref[0], w1_ref[...],
                                (((1,), (1,)), ((), ())),
                                preferred_element_type=jnp.float32)
    sc = sc + pedot + ba_ref[0, 0]

    pos = jax.lax.broadcasted_iota(jnp.int32, (1, seq), 1)
    mask = jnp.zeros((1, seq), dtype=jnp.bool_)
    for t in range(t_spans):
        s0 = spans_ref[b * 2 * t_spans + 2 * t]
        e0 = spans_ref[b * 2 * t_spans + 2 * t + 1]
        mask = mask | ((pos >= s0) & (pos < e0))

    m = jnp.max(jnp.where(mask, sc, -jnp.inf))
    m_safe = jnp.where(jnp.isfinite(m), m, 0.0)
    e = jnp.where(mask, jnp.exp(sc - m_safe), 0.0)
    d = jnp.sum(e)
    probs = jnp.where(d > 0, e / jnp.maximum(d, 1e-30), 0.0)
    probs_ref[...] = probs[None]
    pooled0 = jnp.dot(probs, hid0, preferred_element_type=jnp.float32)
    pooled1 = jnp.dot(probs, hid1, preferred_element_type=jnp.float32)
    if with_proj:
        proj = jnp.tanh(
            jnp.dot(pooled0, wout_ref[:hh, :],
                    preferred_element_type=jnp.float32)
            + jnp.dot(pooled1, wout_ref[hh:, :],
                      preferred_element_type=jnp.float32)
            + bout_ref[...])
        proj_ref[...] = proj[None]
    else:
        pooled_ref[...] = jnp.concatenate([pooled0, pooled1], axis=1)[None]


def _proj_body(p_ref, w_ref, b_ref, o_ref):
    o_ref[...] = jnp.tanh(
        jnp.dot(p_ref[...], w_ref[...], preferred_element_type=jnp.float32)
        + b_ref[...])


def _attention_pool(hidden, spans, pooled_entities, w1, w2, ba, wout, bout,
                    with_proj):
    b, s, h = hidden.shape
    out = wout.shape[1]
    t_spans = spans.shape[0] // (2 * b)
    if with_proj:
        out_specs = [
            pl.BlockSpec((1, 1, s), lambda i, sp: (i, 0, 0)),
            pl.BlockSpec((1, 1, out), lambda i, sp: (i, 0, 0)),
        ]
        out_shape = [
            jax.ShapeDtypeStruct((b, 1, s), jnp.float32),
            jax.ShapeDtypeStruct((b, 1, out), jnp.float32),
        ]
    else:
        out_specs = [
            pl.BlockSpec((1, 1, s), lambda i, sp: (i, 0, 0)),
            pl.BlockSpec((1, 1, h), lambda i, sp: (i, 0, 0)),
        ]
        out_shape = [
            jax.ShapeDtypeStruct((b, 1, s), jnp.float32),
            jax.ShapeDtypeStruct((b, 1, h), jnp.float32),
        ]
    grid_spec = pltpu.PrefetchScalarGridSpec(
        num_scalar_prefetch=1,
        grid=(b,),
        in_specs=[
            pl.BlockSpec((1, s, h // 2), lambda i, sp: (i, 0, 0)),
            pl.BlockSpec((1, s, h // 2), lambda i, sp: (i, 0, 1)),
            pl.BlockSpec((1, h), lambda i, sp: (0, 0)),
            pl.BlockSpec((1, h), lambda i, sp: (0, 0)),
            pl.BlockSpec((1, 1, h), lambda i, sp: (i, 0, 0)),
            pl.BlockSpec((1, 1), lambda i, sp: (0, 0)),
            pl.BlockSpec((h, out), lambda i, sp: (0, 0)),
            pl.BlockSpec((1, out), lambda i, sp: (0, 0)),
        ],
        out_specs=out_specs,
    )
    return pl.pallas_call(
        functools.partial(_fused_body, t_spans, s, h // 2, with_proj),
        grid_spec=grid_spec,
        out_shape=out_shape,
        compiler_params=pltpu.CompilerParams(
            dimension_semantics=("parallel",)),
    )(spans, hidden, hidden, w1, w2, pooled_entities[:, None, :], ba,
      wout, bout)


def _project(pooled, w_out, b_out):
    b, _ = pooled.shape
    out = w_out.shape[1]
    return pl.pallas_call(
        _proj_body,
        out_shape=jax.ShapeDtypeStruct((b, out), jnp.float32),
    )(pooled, w_out, b_out)


def kernel(hidden, token_idxs, pooled_entities, W_align, b_align, W_out, b_out):
    b, s, h = hidden.shape
    f_ent = token_idxs.shape[0]
    w1 = W_align[:h].reshape(1, h)
    w2 = W_align[h:].reshape(1, h)
    ba = b_align.reshape(1, 1).astype(jnp.float32)
    bout = b_out.reshape(1, -1)
    pooled_list = []
    attn_list = []
    proj = None
    for f in range(f_ent):
        spans = token_idxs[f].astype(jnp.int32).reshape(-1)
        wout_f = W_out[f * h:(f + 1) * h]
        probs, second = _attention_pool(
            hidden, spans, pooled_entities, w1, w2, ba, wout_f, bout,
            with_proj=(f_ent == 1))
        attn_list.append(probs.reshape(b, s, 1))
        if f_ent == 1:
            proj = second[:, 0, :]
        else:
            pooled_list.append(second[:, 0, :])
    if f_ent != 1:
        all_pooled = jnp.concatenate(pooled_list, axis=1)
        proj = _project(all_pooled, W_out, bout)
    return proj, jnp.stack(attn_list, axis=0)


# speedup vs baseline: 1.0355x; 1.0355x over previous
import functools

import jax
import jax.numpy as jnp
from jax.experimental import pallas as pl
from jax.experimental.pallas import tpu as pltpu


def _fused_body(t_spans, seq, with_proj, spans_ref,
                hid_ref, w1_ref, w2_ref, pe_ref, ba_ref, wout_ref, bout_ref,
                *refs):
    if with_proj:
        probs_ref, proj_ref = refs
        pooled_ref = None
    else:
        probs_ref, pooled_ref = refs
        proj_ref = None
    b = pl.program_id(0)
    hid = hid_ref[0]
    sc = jax.lax.dot_general(w2_ref[...], hid, (((1,), (1,)), ((), ())),
                             preferred_element_type=jnp.float32)
    pedot = jax.lax.dot_general(pe_ref[0], w1_ref[...],
                                (((1,), (1,)), ((), ())),
                                preferred_element_type=jnp.float32)
    sc = sc + pedot + ba_ref[0, 0]

    pos = jax.lax.broadcasted_iota(jnp.int32, (1, seq), 1)
    mask = jnp.zeros((1, seq), dtype=jnp.bool_)
    for t in range(t_spans):
        s0 = spans_ref[b * 2 * t_spans + 2 * t]
        e0 = spans_ref[b * 2 * t_spans + 2 * t + 1]
        mask = mask | ((pos >= s0) & (pos < e0))

    m = jnp.max(jnp.where(mask, sc, -jnp.inf))
    m_safe = jnp.where(jnp.isfinite(m), m, 0.0)
    e = jnp.where(mask, jnp.exp(sc - m_safe), 0.0)
    d = jnp.sum(e)
    probs = jnp.where(d > 0, e / jnp.maximum(d, 1e-30), 0.0)
    probs_ref[...] = probs[None]
    pooled = jnp.dot(probs, hid, preferred_element_type=jnp.float32)
    if with_proj:
        proj = jnp.tanh(
            jnp.dot(pooled, wout_ref[...],
                    preferred_element_type=jnp.float32) + bout_ref[...])
        proj_ref[...] = proj[None]
    else:
        pooled_ref[...] = pooled[None]


def _proj_body(p_ref, w_ref, b_ref, o_ref):
    o_ref[...] = jnp.tanh(
        jnp.dot(p_ref[...], w_ref[...], preferred_element_type=jnp.float32)
        + b_ref[...])


def _attention_pool(hidden, spans, pooled_entities, w1, w2, ba, wout, bout,
                    with_proj):
    b, s, h = hidden.shape
    out = wout.shape[1]
    t_spans = spans.shape[0] // (2 * b)
    if with_proj:
        out_specs = [
            pl.BlockSpec((1, 1, s), lambda i, sp: (i, 0, 0)),
            pl.BlockSpec((1, 1, out), lambda i, sp: (i, 0, 0)),
        ]
        out_shape = [
            jax.ShapeDtypeStruct((b, 1, s), jnp.float32),
            jax.ShapeDtypeStruct((b, 1, out), jnp.float32),
        ]
    else:
        out_specs = [
            pl.BlockSpec((1, 1, s), lambda i, sp: (i, 0, 0)),
            pl.BlockSpec((1, 1, h), lambda i, sp: (i, 0, 0)),
        ]
        out_shape = [
            jax.ShapeDtypeStruct((b, 1, s), jnp.float32),
            jax.ShapeDtypeStruct((b, 1, h), jnp.float32),
        ]
    grid_spec = pltpu.PrefetchScalarGridSpec(
        num_scalar_prefetch=1,
        grid=(b,),
        in_specs=[
            pl.BlockSpec((1, s, h), lambda i, sp: (i, 0, 0)),
            pl.BlockSpec((1, h), lambda i, sp: (0, 0)),
            pl.BlockSpec((1, h), lambda i, sp: (0, 0)),
            pl.BlockSpec((1, 1, h), lambda i, sp: (i, 0, 0)),
            pl.BlockSpec((1, 1), lambda i, sp: (0, 0)),
            pl.BlockSpec((h, out), lambda i, sp: (0, 0)),
            pl.BlockSpec((1, out), lambda i, sp: (0, 0)),
        ],
        out_specs=out_specs,
    )
    return pl.pallas_call(
        functools.partial(_fused_body, t_spans, s, with_proj),
        grid_spec=grid_spec,
        out_shape=out_shape,
        compiler_params=pltpu.CompilerParams(
            dimension_semantics=("parallel",)),
    )(spans, hidden, w1, w2, pooled_entities[:, None, :], ba, wout, bout)


def _project(pooled, w_out, b_out):
    b, _ = pooled.shape
    out = w_out.shape[1]
    return pl.pallas_call(
        _proj_body,
        out_shape=jax.ShapeDtypeStruct((b, out), jnp.float32),
    )(pooled, w_out, b_out)


def kernel(hidden, token_idxs, pooled_entities, W_align, b_align, W_out, b_out):
    b, s, h = hidden.shape
    f_ent = token_idxs.shape[0]
    w1 = W_align[:h].reshape(1, h)
    w2 = W_align[h:].reshape(1, h)
    ba = b_align.reshape(1, 1).astype(jnp.float32)
    bout = b_out.reshape(1, -1)
    pooled_list = []
    attn_list = []
    proj = None
    for f in range(f_ent):
        spans = token_idxs[f].astype(jnp.int32).reshape(-1)
        wout_f = W_out[f * h:(f + 1) * h]
        probs, second = _attention_pool(
            hidden, spans, pooled_entities, w1, w2, ba, wout_f, bout,
            with_proj=(f_ent == 1))
        attn_list.append(probs.reshape(b, s, 1))
        if f_ent == 1:
            proj = second[:, 0, :]
        else:
            pooled_list.append(second[:, 0, :])
    if f_ent != 1:
        all_pooled = jnp.concatenate(pooled_list, axis=1)
        proj = _project(all_pooled, W_out, bout)
    return proj, jnp.stack(attn_list, axis=0)
